# baseline (speedup 1.0000x reference)
import functools

import jax
import jax.numpy as jnp
from jax import lax
from jax.experimental import pallas as pl
from jax.experimental.pallas import tpu as pltpu
from jax.experimental.pallas import tpu_sc as plsc

_IN_FEATURES = 4096
_OUT_FEATURES = 4096
_IN_GROUP = 8
_NUM_CB = 2
_CB_SIZE = 256
_NIG = _IN_FEATURES // _IN_GROUP

_NW = 32
_ROWS_PER_W = _OUT_FEATURES // _NW
_SLAB = 8
_NSTEPS = _ROWS_PER_W // _SLAB
_NBLK = _IN_FEATURES // 128


def _dequant_body(
    codes_hbm, tab_hbm, w_hbm, cbuf0, cbuf1, tab_v, obuf0, obuf1,
    csem0, csem1, osem0, osem1,
):
    wid = lax.axis_index("s") * 2 + lax.axis_index("c")
    row0 = wid * _ROWS_PER_W

    pltpu.sync_copy(tab_hbm, tab_v)

    lane = lax.iota(jnp.int32, 16)
    zeros = lane * 0
    pos8 = lane * 8

    def codes_slab(s):
        return codes_hbm.at[pl.ds(row0 + s * _SLAB, _SLAB)]

    def w_slab(s):
        off = (row0 + s * _SLAB) * _IN_FEATURES
        return w_hbm.at[pl.ds(off, _SLAB * _IN_FEATURES)]

    def compute_slab(cb, ob):
        @plsc.parallel_loop(0, _SLAB, unroll=2)
        def row(r):
            rsplat = zeros + r
            rcol = r * 128
            for b in range(_NBLK):
                cst = (b // 8) * 256 + (b % 8) * 16
                c0 = plsc.load_gather(cb, [rsplat, lane + cst])
                c1 = plsc.load_gather(cb, [rsplat, lane + (cst + 128)])
                c0 = c0 * 8
                c1 = c1 * 8 + (_CB_SIZE * _IN_GROUP)
                posb = pos8 + (b * 1024) + rcol
                for j in range(_IN_GROUP):
                    v0 = plsc.load_gather(tab_v, [c0 + j])
                    v1 = plsc.load_gather(tab_v, [c1 + j])
                    plsc.store_scatter(ob, [posb + j], v0 + v1)

    pltpu.async_copy(codes_slab(0), cbuf0, csem0)

    def step(si, carry):
        for b in range(2):
            s = si * 2 + b
            cb = cbuf0 if b == 0 else cbuf1
            ob = obuf0 if b == 0 else obuf1
            csem = csem0 if b == 0 else csem1
            osem = osem0 if b == 0 else osem1
            ncb = cbuf1 if b == 0 else cbuf0
            ncsem = csem1 if b == 0 else csem0

            pltpu.make_async_copy(codes_slab(s), cb, csem).wait()

            @pl.when(s < _NSTEPS - 1)
            def _():
                pltpu.async_copy(codes_slab(s + 1), ncb, ncsem)

            @pl.when(s >= 2)
            def _():
                pltpu.make_async_copy(ob, w_slab(s - 2), osem).wait()

            compute_slab(cb, ob)
            pltpu.async_copy(ob, w_slab(s), osem)
        return carry

    lax.fori_loop(0, _NSTEPS // 2, step, 0)
    pltpu.make_async_copy(obuf0, w_slab(_NSTEPS - 2), osem0).wait()
    pltpu.make_async_copy(obuf1, w_slab(_NSTEPS - 1), osem1).wait()


def _sc_dequant(codes, tab):
    mesh = plsc.VectorSubcoreMesh(
        core_axis_name="c", subcore_axis_name="s", num_cores=2, num_subcores=16
    )
    return pl.kernel(
        _dequant_body,
        out_type=jax.ShapeDtypeStruct((_OUT_FEATURES * _IN_FEATURES,), jnp.float32),
        mesh=mesh,
        scratch_types=[
            pltpu.VMEM((_SLAB, _NIG * _NUM_CB), jnp.int32),
            pltpu.VMEM((_SLAB, _NIG * _NUM_CB), jnp.int32),
            pltpu.VMEM((_NUM_CB * _CB_SIZE * _IN_GROUP,), jnp.float32),
            pltpu.VMEM((_SLAB * _IN_FEATURES,), jnp.float32),
            pltpu.VMEM((_SLAB * _IN_FEATURES,), jnp.float32),
            pltpu.SemaphoreType.DMA,
            pltpu.SemaphoreType.DMA,
            pltpu.SemaphoreType.DMA,
            pltpu.SemaphoreType.DMA,
        ],
        compiler_params=pltpu.CompilerParams(
            needs_layout_passes=False, use_tc_tiling_on_sc=False
        ),
    )(codes, tab)


def _gemm_kernel(x_ref, w_ref, s_ref, b_ref, o_ref, acc_ref, *, nk, bn, bk):
    k = pl.program_id(2)

    @pl.when(k == 0)
    def _():
        acc_ref[...] = jnp.zeros_like(acc_ref)

    xb = x_ref[...].astype(jnp.bfloat16)
    wb = jnp.swapaxes(w_ref[...], 1, 2).reshape(bn, bk).astype(jnp.bfloat16)
    acc_ref[...] += lax.dot_general(
        xb, wb, (((1,), (1,)), ((), ())), preferred_element_type=jnp.float32
    )

    @pl.when(k == nk - 1)
    def _():
        o_ref[...] = acc_ref[...] * s_ref[...] + b_ref[...]


def _tc_gemm(x, wt, scales, bias, bm=1024, bn=1024, bk=512):
    m, k = x.shape
    n = _OUT_FEATURES
    nk = k // bk
    grid = (m // bm, n // bn, nk)
    return pl.pallas_call(
        functools.partial(_gemm_kernel, nk=nk, bn=bn, bk=bk),
        grid=grid,
        in_specs=[
            pl.BlockSpec((bm, bk), lambda i, j, kk: (i, kk)),
            pl.BlockSpec((bn // 8, bk // 128, 8, 128), lambda i, j, kk: (j, kk, 0, 0)),
            pl.BlockSpec((1, bn), lambda i, j, kk: (0, j)),
            pl.BlockSpec((1, bn), lambda i, j, kk: (0, j)),
        ],
        out_specs=pl.BlockSpec((bm, bn), lambda i, j, kk: (i, j)),
        out_shape=jax.ShapeDtypeStruct((m, n), jnp.float32),
        scratch_shapes=[pltpu.VMEM((bm, bn), jnp.float32)],
        compiler_params=pltpu.CompilerParams(
            dimension_semantics=("parallel", "parallel", "arbitrary"),
        ),
    )(x, wt, scales.reshape(1, n), bias.reshape(1, n))


def kernel(input, codes, codebooks, scales, bias):
    b, s, f = input.shape
    x = input.reshape(b * s, f)
    tab = codebooks.reshape(-1)
    codes_sc = (
        codes.reshape(_OUT_FEATURES, _NIG // 128, 128, _NUM_CB)
        .transpose(0, 1, 3, 2)
        .reshape(_OUT_FEATURES, _NIG * _NUM_CB)
    )
    w_flat = _sc_dequant(codes_sc, tab)
    wt = w_flat.reshape(_OUT_FEATURES // 8, _NBLK, 8, 128)
    out = _tc_gemm(x, wt, scales.reshape(-1), bias)
    return out.reshape(b, s, _OUT_FEATURES)


# speedup vs baseline: 1.0342x; 1.0342x over previous
import functools

import jax
import jax.numpy as jnp
from jax import lax
from jax.experimental import pallas as pl
from jax.experimental.pallas import tpu as pltpu
from jax.experimental.pallas import tpu_sc as plsc

_IN_FEATURES = 4096
_OUT_FEATURES = 4096
_IN_GROUP = 8
_NUM_CB = 2
_CB_SIZE = 256
_NIG = _IN_FEATURES // _IN_GROUP

_NW = 32
_ROWS_PER_W = _OUT_FEATURES // _NW
_SLAB = 8
_NSTEPS = _ROWS_PER_W // _SLAB
_NBLK = _IN_FEATURES // 128


def _dequant_body(
    codes_hbm, tab_hbm, w_hbm, cbuf0, cbuf1, tab_v, obuf0, obuf1,
    csem0, csem1, osem0, osem1,
):
    wid = lax.axis_index("s") * 2 + lax.axis_index("c")
    row0 = wid * _ROWS_PER_W

    pltpu.sync_copy(tab_hbm, tab_v)

    lane = lax.iota(jnp.int32, 16)
    zeros = lane * 0
    pos8 = lane * 8

    def codes_slab(s):
        return codes_hbm.at[pl.ds(row0 + s * _SLAB, _SLAB)]

    def w_slab(s):
        off = (row0 + s * _SLAB) * _IN_FEATURES
        return w_hbm.at[pl.ds(off, _SLAB * _IN_FEATURES)]

    def compute_slab(cb, ob):
        @plsc.parallel_loop(0, _SLAB)
        def row(r):
            rsplat = zeros + r
            rcol = r * 128
            for b in range(_NBLK):
                cst = (b // 8) * 256 + (b % 8) * 16
                c0 = plsc.load_gather(cb, [rsplat, lane + cst])
                c1 = plsc.load_gather(cb, [rsplat, lane + (cst + 128)])
                c0 = c0 * 8
                c1 = c1 * 8 + (_CB_SIZE * _IN_GROUP)
                posb = pos8 + (b * 1024) + rcol
                for j in range(_IN_GROUP):
                    v0 = plsc.load_gather(tab_v, [c0 + j])
                    v1 = plsc.load_gather(tab_v, [c1 + j])
                    plsc.store_scatter(ob, [posb + j], v0 + v1)

    pltpu.async_copy(codes_slab(0), cbuf0, csem0)

    def step(si, carry):
        for b in range(2):
            s = si * 2 + b
            cb = cbuf0 if b == 0 else cbuf1
            ob = obuf0 if b == 0 else obuf1
            csem = csem0 if b == 0 else csem1
            osem = osem0 if b == 0 else osem1
            ncb = cbuf1 if b == 0 else cbuf0
            ncsem = csem1 if b == 0 else csem0

            pltpu.make_async_copy(codes_slab(s), cb, csem).wait()

            @pl.when(s < _NSTEPS - 1)
            def _():
                pltpu.async_copy(codes_slab(s + 1), ncb, ncsem)

            @pl.when(s >= 2)
            def _():
                pltpu.make_async_copy(ob, w_slab(s - 2), osem).wait()

            compute_slab(cb, ob)
            pltpu.async_copy(ob, w_slab(s), osem)
        return carry

    lax.fori_loop(0, _NSTEPS // 2, step, 0)
    pltpu.make_async_copy(obuf0, w_slab(_NSTEPS - 2), osem0).wait()
    pltpu.make_async_copy(obuf1, w_slab(_NSTEPS - 1), osem1).wait()


def _sc_dequant(codes, tab):
    mesh = plsc.VectorSubcoreMesh(
        core_axis_name="c", subcore_axis_name="s", num_cores=2, num_subcores=16
    )
    return pl.kernel(
        _dequant_body,
        out_type=jax.ShapeDtypeStruct((_OUT_FEATURES * _IN_FEATURES,), jnp.float32),
        mesh=mesh,
        scratch_types=[
            pltpu.VMEM((_SLAB, _NIG * _NUM_CB), jnp.int32),
            pltpu.VMEM((_SLAB, _NIG * _NUM_CB), jnp.int32),
            pltpu.VMEM((_NUM_CB * _CB_SIZE * _IN_GROUP,), jnp.float32),
            pltpu.VMEM((_SLAB * _IN_FEATURES,), jnp.float32),
            pltpu.VMEM((_SLAB * _IN_FEATURES,), jnp.float32),
            pltpu.SemaphoreType.DMA,
            pltpu.SemaphoreType.DMA,
            pltpu.SemaphoreType.DMA,
            pltpu.SemaphoreType.DMA,
        ],
        compiler_params=pltpu.CompilerParams(
            needs_layout_passes=False, use_tc_tiling_on_sc=False
        ),
    )(codes, tab)


def _gemm_kernel(x_ref, w_ref, s_ref, b_ref, o_ref, acc_ref, *, nk, bn, bk):
    k = pl.program_id(2)

    @pl.when(k == 0)
    def _():
        acc_ref[...] = jnp.zeros_like(acc_ref)

    xb = x_ref[...].astype(jnp.bfloat16)
    wb = jnp.swapaxes(w_ref[...], 1, 2).reshape(bn, bk).astype(jnp.bfloat16)
    acc_ref[...] += lax.dot_general(
        xb, wb, (((1,), (1,)), ((), ())), preferred_element_type=jnp.float32
    )

    @pl.when(k == nk - 1)
    def _():
        o_ref[...] = acc_ref[...] * s_ref[...] + b_ref[...]


def _tc_gemm(x, wt, scales, bias, bm=1024, bn=1024, bk=512):
    m, k = x.shape
    n = _OUT_FEATURES
    nk = k // bk
    grid = (m // bm, n // bn, nk)
    return pl.pallas_call(
        functools.partial(_gemm_kernel, nk=nk, bn=bn, bk=bk),
        grid=grid,
        in_specs=[
            pl.BlockSpec((bm, bk), lambda i, j, kk: (i, kk)),
            pl.BlockSpec((bn // 8, bk // 128, 8, 128), lambda i, j, kk: (j, kk, 0, 0)),
            pl.BlockSpec((1, bn), lambda i, j, kk: (0, j)),
            pl.BlockSpec((1, bn), lambda i, j, kk: (0, j)),
        ],
        out_specs=pl.BlockSpec((bm, bn), lambda i, j, kk: (i, j)),
        out_shape=jax.ShapeDtypeStruct((m, n), jnp.float32),
        scratch_shapes=[pltpu.VMEM((bm, bn), jnp.float32)],
        compiler_params=pltpu.CompilerParams(
            dimension_semantics=("parallel", "parallel", "arbitrary"),
        ),
    )(x, wt, scales.reshape(1, n), bias.reshape(1, n))


def kernel(input, codes, codebooks, scales, bias):
    b, s, f = input.shape
    x = input.reshape(b * s, f)
    tab = codebooks.reshape(-1)
    codes_sc = (
        codes.reshape(_OUT_FEATURES, _NIG // 128, 128, _NUM_CB)
        .transpose(0, 1, 3, 2)
        .reshape(_OUT_FEATURES, _NIG * _NUM_CB)
    )
    w_flat = _sc_dequant(codes_sc, tab)
    wt = w_flat.reshape(_OUT_FEATURES // 8, _NBLK, 8, 128)
    out = _tc_gemm(x, wt, scales.reshape(-1), bias)
    return out.reshape(b, s, _OUT_FEATURES)


# speedup vs baseline: 1.1370x; 1.0994x over previous
import functools

import jax
import jax.numpy as jnp
from jax import lax
from jax.experimental import pallas as pl
from jax.experimental.pallas import tpu as pltpu
from jax.experimental.pallas import tpu_sc as plsc

_IN_FEATURES = 4096
_OUT_FEATURES = 4096
_IN_GROUP = 8
_NUM_CB = 2
_CB_SIZE = 256
_NIG = _IN_FEATURES // _IN_GROUP

_NW = 32
_ROWS_PER_W = _OUT_FEATURES // _NW
_SLAB = 8
_NSTEPS = _ROWS_PER_W // _SLAB
_NBLK = _IN_FEATURES // 128


def _dequant_body(
    codes_hbm, tab_hbm, w_hbm, cbuf0, cbuf1, tab_v, obuf0, obuf1,
    csem0, csem1, osem0, osem1,
):
    wid = lax.axis_index("s") * 2 + lax.axis_index("c")
    row0 = wid * _ROWS_PER_W

    pltpu.sync_copy(tab_hbm, tab_v)

    lane = lax.iota(jnp.int32, 16)
    zeros = lane * 0
    pos8 = lane * 8

    def codes_slab(s):
        return codes_hbm.at[pl.ds(row0 + s * _SLAB, _SLAB)]

    def w_slab(s):
        off = (row0 + s * _SLAB) * _IN_FEATURES
        return w_hbm.at[pl.ds(off, _SLAB * _IN_FEATURES)]

    def compute_slab(cb, ob):
        @plsc.parallel_loop(0, _SLAB)
        def row(r):
            rsplat = zeros + r
            rcol = r * 128
            for b in range(_NBLK):
                cst = (b // 8) * 256 + (b % 8) * 16
                c0 = plsc.load_gather(cb, [rsplat, lane + cst])
                c1 = plsc.load_gather(cb, [rsplat, lane + (cst + 128)])
                c0 = c0 * 8
                c1 = c1 * 8 + (_CB_SIZE * _IN_GROUP)
                posb = pos8 + (b * 1024) + rcol
                for j in range(_IN_GROUP):
                    v0 = plsc.load_gather(tab_v, [c0 + j])
                    v1 = plsc.load_gather(tab_v, [c1 + j])
                    plsc.store_scatter(ob, [posb + j], v0 + v1)

    pltpu.async_copy(codes_slab(0), cbuf0, csem0)

    def step(si, carry):
        for b in range(2):
            s = si * 2 + b
            cb = cbuf0 if b == 0 else cbuf1
            ob = obuf0 if b == 0 else obuf1
            csem = csem0 if b == 0 else csem1
            osem = osem0 if b == 0 else osem1
            ncb = cbuf1 if b == 0 else cbuf0
            ncsem = csem1 if b == 0 else csem0

            pltpu.make_async_copy(codes_slab(s), cb, csem).wait()

            @pl.when(s < _NSTEPS - 1)
            def _():
                pltpu.async_copy(codes_slab(s + 1), ncb, ncsem)

            @pl.when(s >= 2)
            def _():
                pltpu.make_async_copy(ob, w_slab(s - 2), osem).wait()

            compute_slab(cb, ob)
            pltpu.async_copy(ob, w_slab(s), osem)
        return carry

    lax.fori_loop(0, _NSTEPS // 2, step, 0)
    pltpu.make_async_copy(obuf0, w_slab(_NSTEPS - 2), osem0).wait()
    pltpu.make_async_copy(obuf1, w_slab(_NSTEPS - 1), osem1).wait()


def _sc_dequant(codes, tab):
    mesh = plsc.VectorSubcoreMesh(
        core_axis_name="c", subcore_axis_name="s", num_cores=2, num_subcores=16
    )
    return pl.kernel(
        _dequant_body,
        out_type=jax.ShapeDtypeStruct((_OUT_FEATURES * _IN_FEATURES,), jnp.float32),
        mesh=mesh,
        scratch_types=[
            pltpu.VMEM((_SLAB, _NIG * _NUM_CB), jnp.int32),
            pltpu.VMEM((_SLAB, _NIG * _NUM_CB), jnp.int32),
            pltpu.VMEM((_NUM_CB * _CB_SIZE * _IN_GROUP,), jnp.float32),
            pltpu.VMEM((_SLAB * _IN_FEATURES,), jnp.float32),
            pltpu.VMEM((_SLAB * _IN_FEATURES,), jnp.float32),
            pltpu.SemaphoreType.DMA,
            pltpu.SemaphoreType.DMA,
            pltpu.SemaphoreType.DMA,
            pltpu.SemaphoreType.DMA,
        ],
        compiler_params=pltpu.CompilerParams(
            needs_layout_passes=False, use_tc_tiling_on_sc=False
        ),
    )(codes, tab)


def _gemm_kernel(x_ref, w_ref, s_ref, b_ref, o_ref, *, nk, bn, bk):
    k = pl.program_id(2)

    @pl.when(k == 0)
    def _():
        o_ref[...] = jnp.zeros_like(o_ref)

    xb = x_ref[...].astype(jnp.bfloat16)
    wb = jnp.swapaxes(w_ref[...], 1, 2).reshape(bn, bk).astype(jnp.bfloat16)
    o_ref[...] += lax.dot_general(
        xb, wb, (((1,), (1,)), ((), ())), preferred_element_type=jnp.float32
    )

    @pl.when(k == nk - 1)
    def _():
        o_ref[...] = o_ref[...] * s_ref[...] + b_ref[...]


def _tc_gemm(x, wt, scales, bias, bm=2048, bn=2048, bk=256):
    m, k = x.shape
    n = _OUT_FEATURES
    nk = k // bk
    grid = (m // bm, n // bn, nk)
    return pl.pallas_call(
        functools.partial(_gemm_kernel, nk=nk, bn=bn, bk=bk),
        grid=grid,
        in_specs=[
            pl.BlockSpec((bm, bk), lambda i, j, kk: (i, kk)),
            pl.BlockSpec((bn // 8, bk // 128, 8, 128), lambda i, j, kk: (j, kk, 0, 0)),
            pl.BlockSpec((1, bn), lambda i, j, kk: (0, j)),
            pl.BlockSpec((1, bn), lambda i, j, kk: (0, j)),
        ],
        out_specs=pl.BlockSpec((bm, bn), lambda i, j, kk: (i, j)),
        out_shape=jax.ShapeDtypeStruct((m, n), jnp.float32),
        compiler_params=pltpu.CompilerParams(
            dimension_semantics=("parallel", "parallel", "arbitrary"),
        ),
    )(x, wt, scales.reshape(1, n), bias.reshape(1, n))


def kernel(input, codes, codebooks, scales, bias):
    b, s, f = input.shape
    x = input.reshape(b * s, f)
    tab = codebooks.reshape(-1)
    codes_sc = (
        codes.reshape(_OUT_FEATURES, _NIG // 128, 128, _NUM_CB)
        .transpose(0, 1, 3, 2)
        .reshape(_OUT_FEATURES, _NIG * _NUM_CB)
    )
    w_flat = _sc_dequant(codes_sc, tab)
    wt = w_flat.reshape(_OUT_FEATURES // 8, _NBLK, 8, 128)
    out = _tc_gemm(x, wt, scales.reshape(-1), bias)
    return out.reshape(b, s, _OUT_FEATURES)


# speedup vs baseline: 1.4104x; 1.2404x over previous
import functools

import jax
import jax.numpy as jnp
from jax import lax
from jax.experimental import pallas as pl
from jax.experimental.pallas import tpu as pltpu
from jax.experimental.pallas import tpu_sc as plsc

_IN_FEATURES = 4096
_OUT_FEATURES = 4096
_IN_GROUP = 8
_NUM_CB = 2
_CB_SIZE = 256
_NIG = _IN_FEATURES // _IN_GROUP

_NW = 32
_ROWS_PER_W = _OUT_FEATURES // _NW
_SLAB = 8
_NSTEPS = _ROWS_PER_W // _SLAB
_NBLK = _IN_FEATURES // 128


_CODES_SLAB = _SLAB * _NIG * _NUM_CB
_W_SLAB = _SLAB * _IN_FEATURES
_TSTRIDE = 513


def _dequant_body(
    codes_hbm, tab_hbm, w_hbm, cbuf, tab_v, obuf, csem0, csem1, osem0, osem1
):
    wid = lax.axis_index("s") * 2 + lax.axis_index("c")
    row0 = wid * _ROWS_PER_W

    pltpu.sync_copy(tab_hbm, tab_v)

    lane = lax.iota(jnp.int32, 16)
    zeros = lane * 0
    pos8 = lane * 8

    def codes_slab(s):
        return codes_hbm.at[pl.ds((row0 + s * _SLAB) * _NIG * _NUM_CB, _CODES_SLAB)]

    def cbuf_half(h):
        return cbuf.at[pl.ds(h * _CODES_SLAB, _CODES_SLAB)]

    def obuf_half(h):
        return obuf.at[pl.ds(h * _W_SLAB, _W_SLAB)]

    def w_slab(s):
        off = (row0 + s * _SLAB) * _IN_FEATURES
        return w_hbm.at[pl.ds(off, _W_SLAB)]

    def compute_slab(cur):
        def row(r, carry2):
            cbase = zeros + (cur * _CODES_SLAB + r * (_NIG * _NUM_CB))
            obase = pos8 + (cur * _W_SLAB + r * 128)
            for b in range(_NBLK):
                cst = (b // 8) * 256 + (b % 8) * 16
                c0 = plsc.load_gather(cbuf, [cbase + (lane + cst)])
                c1 = plsc.load_gather(cbuf, [cbase + (lane + (cst + 128))])
                c1 = c1 + _CB_SIZE
                posb = obase + (b * 1024)
                for j in range(_IN_GROUP):
                    v0 = plsc.load_gather(tab_v, [c0 + j * _TSTRIDE])
                    v1 = plsc.load_gather(tab_v, [c1 + j * _TSTRIDE])
                    plsc.store_scatter(obuf, [posb + j], v0 + v1)
            return carry2

        lax.fori_loop(0, _SLAB, row, 0)

    pltpu.async_copy(codes_slab(0), cbuf_half(0), csem0)

    def step(s, carry):
        cur = lax.rem(s, 2)

        @pl.when(cur == 0)
        def _():
            pltpu.make_async_copy(codes_slab(s), cbuf_half(0), csem0).wait()

            @pl.when(s < _NSTEPS - 1)
            def _():
                pltpu.async_copy(codes_slab(s + 1), cbuf_half(1), csem1)

            @pl.when(s >= 2)
            def _():
                pltpu.make_async_copy(obuf_half(0), w_slab(s - 2), osem0).wait()

        @pl.when(cur == 1)
        def _():
            pltpu.make_async_copy(codes_slab(s), cbuf_half(1), csem1).wait()

            @pl.when(s < _NSTEPS - 1)
            def _():
                pltpu.async_copy(codes_slab(s + 1), cbuf_half(0), csem0)

            @pl.when(s >= 2)
            def _():
                pltpu.make_async_copy(obuf_half(1), w_slab(s - 2), osem1).wait()

        compute_slab(cur)

        @pl.when(cur == 0)
        def _():
            pltpu.async_copy(obuf_half(0), w_slab(s), osem0)

        @pl.when(cur == 1)
        def _():
            pltpu.async_copy(obuf_half(1), w_slab(s), osem1)

        return carry

    lax.fori_loop(0, _NSTEPS, step, 0)
    pltpu.make_async_copy(obuf_half(0), w_slab(_NSTEPS - 2), osem0).wait()
    pltpu.make_async_copy(obuf_half(1), w_slab(_NSTEPS - 1), osem1).wait()


def _sc_dequant(codes_flat, tab_t):
    mesh = plsc.VectorSubcoreMesh(
        core_axis_name="c", subcore_axis_name="s", num_cores=2, num_subcores=16
    )
    return pl.kernel(
        _dequant_body,
        out_type=jax.ShapeDtypeStruct((_OUT_FEATURES * _IN_FEATURES,), jnp.float32),
        mesh=mesh,
        scratch_types=[
            pltpu.VMEM((2 * _CODES_SLAB,), jnp.int32),
            pltpu.VMEM((_IN_GROUP * _TSTRIDE,), jnp.float32),
            pltpu.VMEM((2 * _W_SLAB,), jnp.float32),
            pltpu.SemaphoreType.DMA,
            pltpu.SemaphoreType.DMA,
            pltpu.SemaphoreType.DMA,
            pltpu.SemaphoreType.DMA,
        ],
        compiler_params=pltpu.CompilerParams(
            needs_layout_passes=False, use_tc_tiling_on_sc=False
        ),
    )(codes_flat, tab_t)


def _gemm_kernel(x_ref, w_ref, s_ref, b_ref, o_ref, *, nk, bn, bk):
    k = pl.program_id(2)

    @pl.when(k == 0)
    def _():
        o_ref[...] = jnp.zeros_like(o_ref)

    xb = x_ref[...].astype(jnp.bfloat16)
    wb = jnp.swapaxes(w_ref[...], 1, 2).reshape(bn, bk).astype(jnp.bfloat16)
    o_ref[...] += lax.dot_general(
        xb, wb, (((1,), (1,)), ((), ())), preferred_element_type=jnp.float32
    )

    @pl.when(k == nk - 1)
    def _():
        o_ref[...] = o_ref[...] * s_ref[...] + b_ref[...]


def _tc_gemm(x, wt, scales, bias, bm=2048, bn=2048, bk=256):
    m, k = x.shape
    n = _OUT_FEATURES
    nk = k // bk
    grid = (m // bm, n // bn, nk)
    return pl.pallas_call(
        functools.partial(_gemm_kernel, nk=nk, bn=bn, bk=bk),
        grid=grid,
        in_specs=[
            pl.BlockSpec((bm, bk), lambda i, j, kk: (i, kk)),
            pl.BlockSpec((bn // 8, bk // 128, 8, 128), lambda i, j, kk: (j, kk, 0, 0)),
            pl.BlockSpec((1, bn), lambda i, j, kk: (0, j)),
            pl.BlockSpec((1, bn), lambda i, j, kk: (0, j)),
        ],
        out_specs=pl.BlockSpec((bm, bn), lambda i, j, kk: (i, j)),
        out_shape=jax.ShapeDtypeStruct((m, n), jnp.float32),
        compiler_params=pltpu.CompilerParams(
            dimension_semantics=("parallel", "parallel", "arbitrary"),
        ),
    )(x, wt, scales.reshape(1, n), bias.reshape(1, n))


def kernel(input, codes, codebooks, scales, bias):
    b, s, f = input.shape
    x = input.reshape(b * s, f)
    tab_t = jnp.pad(
        codebooks.reshape(_NUM_CB * _CB_SIZE, _IN_GROUP).T,
        ((0, 0), (0, _TSTRIDE - _NUM_CB * _CB_SIZE)),
    ).reshape(-1)
    codes_sc = (
        codes.reshape(_OUT_FEATURES, _NIG // 128, 128, _NUM_CB)
        .transpose(0, 1, 3, 2)
        .reshape(-1)
    )
    w_flat = _sc_dequant(codes_sc, tab_t)
    wt = w_flat.reshape(_OUT_FEATURES // 8, _NBLK, 8, 128)
    out = _tc_gemm(x, wt, scales.reshape(-1), bias)
    return out.reshape(b, s, _OUT_FEATURES)


# speedup vs baseline: 1.5764x; 1.1177x over previous
import functools

import jax
import jax.numpy as jnp
from jax import lax
from jax.experimental import pallas as pl
from jax.experimental.pallas import tpu as pltpu
from jax.experimental.pallas import tpu_sc as plsc

_IN_FEATURES = 4096
_OUT_FEATURES = 4096
_IN_GROUP = 8
_NUM_CB = 2
_CB_SIZE = 256
_NIG = _IN_FEATURES // _IN_GROUP

_NW = 32
_NCHUNK = 2
_CHUNK_ROWS = _OUT_FEATURES // _NCHUNK
_ROWS_PER_W = _CHUNK_ROWS // _NW
_SLAB = 8
_NSTEPS = _ROWS_PER_W // _SLAB
_NBLK = _IN_FEATURES // 128


_CODES_SLAB = _SLAB * _NIG * _NUM_CB
_W_SLAB = _SLAB * _IN_FEATURES
_TSTRIDE = 513


def _dequant_body(
    codes_hbm, tab_hbm, w_hbm, cbuf, tab_v, obuf, csem0, csem1, osem0, osem1,
    *, row_base
):
    wid = lax.axis_index("s") * 2 + lax.axis_index("c")
    row0 = wid * _ROWS_PER_W

    pltpu.sync_copy(tab_hbm, tab_v)

    lane = lax.iota(jnp.int32, 16)
    zeros = lane * 0
    pos8 = lane * 8

    def codes_slab(s):
        off = (row_base + row0 + s * _SLAB) * _NIG * _NUM_CB
        return codes_hbm.at[pl.ds(off, _CODES_SLAB)]

    def cbuf_half(h):
        return cbuf.at[pl.ds(h * _CODES_SLAB, _CODES_SLAB)]

    def obuf_half(h):
        return obuf.at[pl.ds(h * _W_SLAB, _W_SLAB)]

    def w_slab(s):
        off = (row0 + s * _SLAB) * _IN_FEATURES
        return w_hbm.at[pl.ds(off, _W_SLAB)]

    def compute_slab(cur):
        def row(r, carry2):
            cbase = zeros + (cur * _CODES_SLAB + r * (_NIG * _NUM_CB))
            obase = pos8 + (cur * _W_SLAB + r * 128)
            for b in range(_NBLK):
                cst = (b // 8) * 256 + (b % 8) * 16
                c0 = plsc.load_gather(cbuf, [cbase + (lane + cst)])
                c1 = plsc.load_gather(cbuf, [cbase + (lane + (cst + 128))])
                c1 = c1 + _CB_SIZE
                posb = obase + (b * 1024)
                for j in range(_IN_GROUP):
                    v0 = plsc.load_gather(tab_v, [c0 + j * _TSTRIDE])
                    v1 = plsc.load_gather(tab_v, [c1 + j * _TSTRIDE])
                    plsc.store_scatter(obuf, [posb + j], v0 + v1)
            return carry2

        lax.fori_loop(0, _SLAB, row, 0)

    pltpu.async_copy(codes_slab(0), cbuf_half(0), csem0)

    def step(s, carry):
        cur = lax.rem(s, 2)

        @pl.when(cur == 0)
        def _():
            pltpu.make_async_copy(codes_slab(s), cbuf_half(0), csem0).wait()

            @pl.when(s < _NSTEPS - 1)
            def _():
                pltpu.async_copy(codes_slab(s + 1), cbuf_half(1), csem1)

            @pl.when(s >= 2)
            def _():
                pltpu.make_async_copy(obuf_half(0), w_slab(s - 2), osem0).wait()

        @pl.when(cur == 1)
        def _():
            pltpu.make_async_copy(codes_slab(s), cbuf_half(1), csem1).wait()

            @pl.when(s < _NSTEPS - 1)
            def _():
                pltpu.async_copy(codes_slab(s + 1), cbuf_half(0), csem0)

            @pl.when(s >= 2)
            def _():
                pltpu.make_async_copy(obuf_half(1), w_slab(s - 2), osem1).wait()

        compute_slab(cur)

        @pl.when(cur == 0)
        def _():
            pltpu.async_copy(obuf_half(0), w_slab(s), osem0)

        @pl.when(cur == 1)
        def _():
            pltpu.async_copy(obuf_half(1), w_slab(s), osem1)

        return carry

    lax.fori_loop(0, _NSTEPS, step, 0)
    pltpu.make_async_copy(obuf_half(0), w_slab(_NSTEPS - 2), osem0).wait()
    pltpu.make_async_copy(obuf_half(1), w_slab(_NSTEPS - 1), osem1).wait()


def _sc_dequant(codes_flat, tab_t, row_base):
    mesh = plsc.VectorSubcoreMesh(
        core_axis_name="c", subcore_axis_name="s", num_cores=2, num_subcores=16
    )
    return pl.kernel(
        functools.partial(_dequant_body, row_base=row_base),
        out_type=jax.ShapeDtypeStruct((_CHUNK_ROWS * _IN_FEATURES,), jnp.float32),
        mesh=mesh,
        scratch_types=[
            pltpu.VMEM((2 * _CODES_SLAB,), jnp.int32),
            pltpu.VMEM((_IN_GROUP * _TSTRIDE,), jnp.float32),
            pltpu.VMEM((2 * _W_SLAB,), jnp.float32),
            pltpu.SemaphoreType.DMA,
            pltpu.SemaphoreType.DMA,
            pltpu.SemaphoreType.DMA,
            pltpu.SemaphoreType.DMA,
        ],
        compiler_params=pltpu.CompilerParams(
            needs_layout_passes=False, use_tc_tiling_on_sc=False
        ),
    )(codes_flat, tab_t)


def _gemm_kernel(x_ref, w_ref, s_ref, b_ref, prev_ref, o_ref, *, nk, bn, bk):
    del prev_ref
    k = pl.program_id(2)

    @pl.when(k == 0)
    def _():
        o_ref[...] = jnp.zeros_like(o_ref)

    xb = x_ref[...].astype(jnp.bfloat16)
    wb = jnp.swapaxes(w_ref[...], 1, 2).reshape(bn, bk).astype(jnp.bfloat16)
    o_ref[...] += lax.dot_general(
        xb, wb, (((1,), (1,)), ((), ())), preferred_element_type=jnp.float32
    )

    @pl.when(k == nk - 1)
    def _():
        o_ref[...] = o_ref[...] * s_ref[...] + b_ref[...]


def _tc_gemm_half(x, wt, s_half, b_half, prev, n_off, bm=2048, bk=256):
    m, k = x.shape
    bn = _CHUNK_ROWS
    nk = k // bk
    grid = (m // bm, 1, nk)
    return pl.pallas_call(
        functools.partial(_gemm_kernel, nk=nk, bn=bn, bk=bk),
        grid=grid,
        in_specs=[
            pl.BlockSpec((bm, bk), lambda i, j, kk: (i, kk)),
            pl.BlockSpec((bn // 8, bk // 128, 8, 128), lambda i, j, kk: (0, kk, 0, 0)),
            pl.BlockSpec((1, bn), lambda i, j, kk: (0, 0)),
            pl.BlockSpec((1, bn), lambda i, j, kk: (0, 0)),
            pl.BlockSpec(memory_space=pl.ANY),
        ],
        out_specs=pl.BlockSpec((bm, bn), lambda i, j, kk: (i, n_off)),
        out_shape=jax.ShapeDtypeStruct((m, _OUT_FEATURES), jnp.float32),
        input_output_aliases={4: 0},
        compiler_params=pltpu.CompilerParams(
            dimension_semantics=("parallel", "parallel", "arbitrary"),
        ),
    )(x, wt, s_half, b_half, prev)


def kernel(input, codes, codebooks, scales, bias):
    b, s, f = input.shape
    m = b * s
    x = input.reshape(m, f)
    tab_t = jnp.pad(
        codebooks.reshape(_NUM_CB * _CB_SIZE, _IN_GROUP).T,
        ((0, 0), (0, _TSTRIDE - _NUM_CB * _CB_SIZE)),
    ).reshape(-1)
    codes_sc = (
        codes.reshape(_OUT_FEATURES, _NIG // 128, 128, _NUM_CB)
        .transpose(0, 1, 3, 2)
        .reshape(-1)
    )
    sv = scales.reshape(1, _OUT_FEATURES)
    bv = bias.reshape(1, _OUT_FEATURES)
    out = None
    for h in range(_NCHUNK):
        w_flat = _sc_dequant(codes_sc, tab_t, h * _CHUNK_ROWS)
        wt = w_flat.reshape(_CHUNK_ROWS // 8, _NBLK, 8, 128)
        lo = h * _CHUNK_ROWS
        s_half = lax.slice(sv, (0, lo), (1, lo + _CHUNK_ROWS))
        b_half = lax.slice(bv, (0, lo), (1, lo + _CHUNK_ROWS))
        if out is None:
            out = jnp.zeros((m, _OUT_FEATURES), jnp.float32)
        out = _tc_gemm_half(x, wt, s_half, b_half, out, h)
    return out.reshape(b, s, _OUT_FEATURES)


# speedup vs baseline: 1.6031x; 1.0169x over previous
import functools

import jax
import jax.numpy as jnp
from jax import lax
from jax.experimental import pallas as pl
from jax.experimental.pallas import tpu as pltpu
from jax.experimental.pallas import tpu_sc as plsc

_IN_FEATURES = 4096
_OUT_FEATURES = 4096
_IN_GROUP = 8
_NUM_CB = 2
_CB_SIZE = 256
_NIG = _IN_FEATURES // _IN_GROUP

_NW = 32
_NCHUNK = 2
_CHUNK_ROWS = _OUT_FEATURES // _NCHUNK
_ROWS_PER_W = _CHUNK_ROWS // _NW
_SLAB = 8
_NSTEPS = _ROWS_PER_W // _SLAB
_NBLK = _IN_FEATURES // 128


_CODES_SLAB = _SLAB * _NIG * _NUM_CB
_W_SLAB = _SLAB * _IN_FEATURES
_TSTRIDE = 513


def _dequant_body(
    codes_hbm, tab_hbm, w_hbm, cbuf, tab_v, obuf, csem0, csem1, osem0, osem1,
    *, row_base
):
    wid = lax.axis_index("s") * 2 + lax.axis_index("c")
    row0 = wid * _ROWS_PER_W

    pltpu.sync_copy(tab_hbm, tab_v)

    lane = lax.iota(jnp.int32, 16)
    zeros = lane * 0
    pos8 = lane * 8

    def codes_slab(s):
        off = (row_base + row0 + s * _SLAB) * _NIG * _NUM_CB
        return codes_hbm.at[pl.ds(off, _CODES_SLAB)]

    def cbuf_half(h):
        return cbuf.at[pl.ds(h * _CODES_SLAB, _CODES_SLAB)]

    def obuf_half(h):
        return obuf.at[pl.ds(h * _W_SLAB, _W_SLAB)]

    def w_slab(s):
        off = (row0 + s * _SLAB) * _IN_FEATURES
        return w_hbm.at[pl.ds(off, _W_SLAB)]

    def compute_slab(cur):
        def row(r, carry2):
            cbase = zeros + (cur * _CODES_SLAB + r * (_NIG * _NUM_CB))
            obase = pos8 + (cur * _W_SLAB + r * 128)
            for b in range(_NBLK):
                cst = (b // 8) * 256 + (b % 8) * 16
                c0 = plsc.load_gather(cbuf, [cbase + (lane + cst)])
                c1 = plsc.load_gather(cbuf, [cbase + (lane + (cst + 128))])
                c1 = c1 + _CB_SIZE
                posb = obase + (b * 1024)
                for j in range(_IN_GROUP):
                    v0 = plsc.load_gather(tab_v, [c0 + j * _TSTRIDE])
                    v1 = plsc.load_gather(tab_v, [c1 + j * _TSTRIDE])
                    plsc.store_scatter(obuf, [posb + j], v0 + v1)
            return carry2

        lax.fori_loop(0, _SLAB, row, 0)

    pltpu.async_copy(codes_slab(0), cbuf_half(0), csem0)

    def step(s, carry):
        cur = lax.rem(s, 2)

        @pl.when(cur == 0)
        def _():
            pltpu.make_async_copy(codes_slab(s), cbuf_half(0), csem0).wait()

            @pl.when(s < _NSTEPS - 1)
            def _():
                pltpu.async_copy(codes_slab(s + 1), cbuf_half(1), csem1)

            @pl.when(s >= 2)
            def _():
                pltpu.make_async_copy(obuf_half(0), w_slab(s - 2), osem0).wait()

        @pl.when(cur == 1)
        def _():
            pltpu.make_async_copy(codes_slab(s), cbuf_half(1), csem1).wait()

            @pl.when(s < _NSTEPS - 1)
            def _():
                pltpu.async_copy(codes_slab(s + 1), cbuf_half(0), csem0)

            @pl.when(s >= 2)
            def _():
                pltpu.make_async_copy(obuf_half(1), w_slab(s - 2), osem1).wait()

        compute_slab(cur)

        @pl.when(cur == 0)
        def _():
            pltpu.async_copy(obuf_half(0), w_slab(s), osem0)

        @pl.when(cur == 1)
        def _():
            pltpu.async_copy(obuf_half(1), w_slab(s), osem1)

        return carry

    lax.fori_loop(0, _NSTEPS, step, 0)
    pltpu.make_async_copy(obuf_half(0), w_slab(_NSTEPS - 2), osem0).wait()
    pltpu.make_async_copy(obuf_half(1), w_slab(_NSTEPS - 1), osem1).wait()


def _sc_dequant(codes_flat, tab_t, row_base):
    mesh = plsc.VectorSubcoreMesh(
        core_axis_name="c", subcore_axis_name="s", num_cores=2, num_subcores=16
    )
    return pl.kernel(
        functools.partial(_dequant_body, row_base=row_base),
        out_type=jax.ShapeDtypeStruct((_CHUNK_ROWS * _IN_FEATURES,), jnp.float32),
        mesh=mesh,
        scratch_types=[
            pltpu.VMEM((2 * _CODES_SLAB,), jnp.int32),
            pltpu.VMEM((_IN_GROUP * _TSTRIDE,), jnp.float32),
            pltpu.VMEM((2 * _W_SLAB,), jnp.float32),
            pltpu.SemaphoreType.DMA,
            pltpu.SemaphoreType.DMA,
            pltpu.SemaphoreType.DMA,
            pltpu.SemaphoreType.DMA,
        ],
        compiler_params=pltpu.CompilerParams(
            needs_layout_passes=False, use_tc_tiling_on_sc=False
        ),
    )(codes_flat, tab_t)


def _gemm_kernel(x_ref, w_ref, s_ref, b_ref, *rest, nk, bn, bk):
    o_ref = rest[-1]
    k = pl.program_id(2)

    @pl.when(k == 0)
    def _():
        o_ref[...] = jnp.zeros_like(o_ref)

    xb = x_ref[...].astype(jnp.bfloat16)
    wb = jnp.swapaxes(w_ref[...], 1, 2).reshape(bn, bk).astype(jnp.bfloat16)
    o_ref[...] += lax.dot_general(
        xb, wb, (((1,), (1,)), ((), ())), preferred_element_type=jnp.float32
    )

    @pl.when(k == nk - 1)
    def _():
        o_ref[...] = o_ref[...] * s_ref[...] + b_ref[...]


def _tc_gemm_half(x, wt, s_half, b_half, prev, n_off, bm=2048, bk=256):
    m, k = x.shape
    bn = _CHUNK_ROWS
    nk = k // bk
    grid = (m // bm, 1, nk)
    in_specs = [
        pl.BlockSpec((bm, bk), lambda i, j, kk: (i, kk)),
        pl.BlockSpec((bn // 8, bk // 128, 8, 128), lambda i, j, kk: (0, kk, 0, 0)),
        pl.BlockSpec((1, bn), lambda i, j, kk: (0, 0)),
        pl.BlockSpec((1, bn), lambda i, j, kk: (0, 0)),
    ]
    args = (x, wt, s_half, b_half)
    aliases = {}
    if prev is not None:
        in_specs.append(pl.BlockSpec(memory_space=pl.ANY))
        args += (prev,)
        aliases = {4: 0}
    return pl.pallas_call(
        functools.partial(_gemm_kernel, nk=nk, bn=bn, bk=bk),
        grid=grid,
        in_specs=in_specs,
        out_specs=pl.BlockSpec((bm, bn), lambda i, j, kk: (i, n_off)),
        out_shape=jax.ShapeDtypeStruct((m, _OUT_FEATURES), jnp.float32),
        input_output_aliases=aliases,
        compiler_params=pltpu.CompilerParams(
            dimension_semantics=("parallel", "parallel", "arbitrary"),
        ),
    )(*args)


def kernel(input, codes, codebooks, scales, bias):
    b, s, f = input.shape
    m = b * s
    x = input.reshape(m, f)
    tab_t = jnp.pad(
        codebooks.reshape(_NUM_CB * _CB_SIZE, _IN_GROUP).T,
        ((0, 0), (0, _TSTRIDE - _NUM_CB * _CB_SIZE)),
    ).reshape(-1)
    codes_sc = (
        codes.reshape(_OUT_FEATURES, _NIG // 128, 128, _NUM_CB)
        .transpose(0, 1, 3, 2)
        .reshape(-1)
    )
    sv = scales.reshape(1, _OUT_FEATURES)
    bv = bias.reshape(1, _OUT_FEATURES)
    out = None
    for h in range(_NCHUNK):
        w_flat = _sc_dequant(codes_sc, tab_t, h * _CHUNK_ROWS)
        wt = w_flat.reshape(_CHUNK_ROWS // 8, _NBLK, 8, 128)
        lo = h * _CHUNK_ROWS
        s_half = lax.slice(sv, (0, lo), (1, lo + _CHUNK_ROWS))
        b_half = lax.slice(bv, (0, lo), (1, lo + _CHUNK_ROWS))
        out = _tc_gemm_half(x, wt, s_half, b_half, out, h)
    return out.reshape(b, s, _OUT_FEATURES)


# speedup vs baseline: 1.6429x; 1.0249x over previous
import functools

import jax
import jax.numpy as jnp
from jax import lax
from jax.experimental import pallas as pl
from jax.experimental.pallas import tpu as pltpu
from jax.experimental.pallas import tpu_sc as plsc

_IN_FEATURES = 4096
_OUT_FEATURES = 4096
_IN_GROUP = 8
_NUM_CB = 2
_CB_SIZE = 256
_NIG = _IN_FEATURES // _IN_GROUP

_NW = 32
_NCHUNK = 4
_CHUNK_ROWS = _OUT_FEATURES // _NCHUNK
_ROWS_PER_W = _CHUNK_ROWS // _NW
_SLAB = 8
_NSTEPS = _ROWS_PER_W // _SLAB
_NBLK = _IN_FEATURES // 128


_CODES_SLAB = _SLAB * _NIG * _NUM_CB
_W_SLAB = _SLAB * _IN_FEATURES
_TSTRIDE = 513


def _dequant_body(
    codes_hbm, tab_hbm, w_hbm, cbuf, tab_v, obuf, csem0, csem1, osem0, osem1,
    *, row_base
):
    wid = lax.axis_index("s") * 2 + lax.axis_index("c")
    row0 = wid * _ROWS_PER_W

    pltpu.sync_copy(tab_hbm, tab_v)

    lane = lax.iota(jnp.int32, 16)
    zeros = lane * 0
    pos8 = lane * 8

    def codes_slab(s):
        off = (row_base + row0 + s * _SLAB) * _NIG * _NUM_CB
        return codes_hbm.at[pl.ds(off, _CODES_SLAB)]

    def cbuf_half(h):
        return cbuf.at[pl.ds(h * _CODES_SLAB, _CODES_SLAB)]

    def obuf_half(h):
        return obuf.at[pl.ds(h * _W_SLAB, _W_SLAB)]

    def w_slab(s):
        off = (row0 + s * _SLAB) * _IN_FEATURES
        return w_hbm.at[pl.ds(off, _W_SLAB)]

    def compute_slab(cur):
        def row(r, carry2):
            cbase = zeros + (cur * _CODES_SLAB + r * (_NIG * _NUM_CB))
            obase = pos8 + (cur * _W_SLAB + r * 128)
            for b in range(_NBLK):
                cst = (b // 8) * 256 + (b % 8) * 16
                c0 = plsc.load_gather(cbuf, [cbase + (lane + cst)])
                c1 = plsc.load_gather(cbuf, [cbase + (lane + (cst + 128))])
                c1 = c1 + _CB_SIZE
                posb = obase + (b * 1024)
                for j in range(_IN_GROUP):
                    v0 = plsc.load_gather(tab_v, [c0 + j * _TSTRIDE])
                    v1 = plsc.load_gather(tab_v, [c1 + j * _TSTRIDE])
                    plsc.store_scatter(obuf, [posb + j], v0 + v1)
            return carry2

        lax.fori_loop(0, _SLAB, row, 0)

    pltpu.async_copy(codes_slab(0), cbuf_half(0), csem0)

    def step(s, carry):
        cur = lax.rem(s, 2)

        @pl.when(cur == 0)
        def _():
            pltpu.make_async_copy(codes_slab(s), cbuf_half(0), csem0).wait()

            @pl.when(s < _NSTEPS - 1)
            def _():
                pltpu.async_copy(codes_slab(s + 1), cbuf_half(1), csem1)

            @pl.when(s >= 2)
            def _():
                pltpu.make_async_copy(obuf_half(0), w_slab(s - 2), osem0).wait()

        @pl.when(cur == 1)
        def _():
            pltpu.make_async_copy(codes_slab(s), cbuf_half(1), csem1).wait()

            @pl.when(s < _NSTEPS - 1)
            def _():
                pltpu.async_copy(codes_slab(s + 1), cbuf_half(0), csem0)

            @pl.when(s >= 2)
            def _():
                pltpu.make_async_copy(obuf_half(1), w_slab(s - 2), osem1).wait()

        compute_slab(cur)

        @pl.when(cur == 0)
        def _():
            pltpu.async_copy(obuf_half(0), w_slab(s), osem0)

        @pl.when(cur == 1)
        def _():
            pltpu.async_copy(obuf_half(1), w_slab(s), osem1)

        return carry

    lax.fori_loop(0, _NSTEPS, step, 0)
    pltpu.make_async_copy(obuf_half(0), w_slab(_NSTEPS - 2), osem0).wait()
    pltpu.make_async_copy(obuf_half(1), w_slab(_NSTEPS - 1), osem1).wait()


def _sc_dequant(codes_flat, tab_t, row_base):
    mesh = plsc.VectorSubcoreMesh(
        core_axis_name="c", subcore_axis_name="s", num_cores=2, num_subcores=16
    )
    return pl.kernel(
        functools.partial(_dequant_body, row_base=row_base),
        out_type=jax.ShapeDtypeStruct((_CHUNK_ROWS * _IN_FEATURES,), jnp.float32),
        mesh=mesh,
        scratch_types=[
            pltpu.VMEM((2 * _CODES_SLAB,), jnp.int32),
            pltpu.VMEM((_IN_GROUP * _TSTRIDE,), jnp.float32),
            pltpu.VMEM((2 * _W_SLAB,), jnp.float32),
            pltpu.SemaphoreType.DMA,
            pltpu.SemaphoreType.DMA,
            pltpu.SemaphoreType.DMA,
            pltpu.SemaphoreType.DMA,
        ],
        compiler_params=pltpu.CompilerParams(
            needs_layout_passes=False, use_tc_tiling_on_sc=False
        ),
    )(codes_flat, tab_t)


def _gemm_kernel(x_ref, w_ref, s_ref, b_ref, *rest, nk, bn, bk):
    o_ref = rest[-1]
    k = pl.program_id(2)

    @pl.when(k == 0)
    def _():
        o_ref[...] = jnp.zeros_like(o_ref)

    xb = x_ref[...].astype(jnp.bfloat16)
    wb = jnp.swapaxes(w_ref[...], 1, 2).reshape(bn, bk).astype(jnp.bfloat16)
    o_ref[...] += lax.dot_general(
        xb, wb, (((1,), (1,)), ((), ())), preferred_element_type=jnp.float32
    )

    @pl.when(k == nk - 1)
    def _():
        o_ref[...] = o_ref[...] * s_ref[...] + b_ref[...]


def _tc_gemm_half(x, wt, s_half, b_half, prev, n_off, bm=2048, bk=256):
    m, k = x.shape
    bn = _CHUNK_ROWS
    nk = k // bk
    grid = (m // bm, 1, nk)
    in_specs = [
        pl.BlockSpec((bm, bk), lambda i, j, kk: (i, kk)),
        pl.BlockSpec((bn // 8, bk // 128, 8, 128), lambda i, j, kk: (0, kk, 0, 0)),
        pl.BlockSpec((1, bn), lambda i, j, kk: (0, 0)),
        pl.BlockSpec((1, bn), lambda i, j, kk: (0, 0)),
    ]
    args = (x, wt, s_half, b_half)
    aliases = {}
    if prev is not None:
        in_specs.append(pl.BlockSpec(memory_space=pl.ANY))
        args += (prev,)
        aliases = {4: 0}
    return pl.pallas_call(
        functools.partial(_gemm_kernel, nk=nk, bn=bn, bk=bk),
        grid=grid,
        in_specs=in_specs,
        out_specs=pl.BlockSpec((bm, bn), lambda i, j, kk: (i, n_off)),
        out_shape=jax.ShapeDtypeStruct((m, _OUT_FEATURES), jnp.float32),
        input_output_aliases=aliases,
        compiler_params=pltpu.CompilerParams(
            dimension_semantics=("parallel", "parallel", "arbitrary"),
        ),
    )(*args)


def kernel(input, codes, codebooks, scales, bias):
    b, s, f = input.shape
    m = b * s
    x = input.reshape(m, f)
    tab_t = jnp.pad(
        codebooks.reshape(_NUM_CB * _CB_SIZE, _IN_GROUP).T,
        ((0, 0), (0, _TSTRIDE - _NUM_CB * _CB_SIZE)),
    ).reshape(-1)
    codes_sc = (
        codes.reshape(_OUT_FEATURES, _NIG // 128, 128, _NUM_CB)
        .transpose(0, 1, 3, 2)
        .reshape(-1)
    )
    sv = scales.reshape(1, _OUT_FEATURES)
    bv = bias.reshape(1, _OUT_FEATURES)
    out = None
    for h in range(_NCHUNK):
        w_flat = _sc_dequant(codes_sc, tab_t, h * _CHUNK_ROWS)
        wt = w_flat.reshape(_CHUNK_ROWS // 8, _NBLK, 8, 128)
        lo = h * _CHUNK_ROWS
        s_half = lax.slice(sv, (0, lo), (1, lo + _CHUNK_ROWS))
        b_half = lax.slice(bv, (0, lo), (1, lo + _CHUNK_ROWS))
        out = _tc_gemm_half(x, wt, s_half, b_half, out, h)
    return out.reshape(b, s, _OUT_FEATURES)
